# baseline (speedup 1.0000x reference)
import functools

import jax
import jax.numpy as jnp
from jax import lax
from jax.experimental import pallas as pl
from jax.experimental.pallas import tpu as pltpu
from jax.experimental.pallas import tpu_sc as plsc

NUM_FEATURES = 4
PER_DIM = 32
NC = 2
NS = 16
NW = NC * NS
L = 16


@functools.cache
def _make_gather(batch: int, vocabs: tuple) -> object:
    bpw = batch // NW
    mesh = plsc.VectorSubcoreMesh(core_axis_name="c", subcore_axis_name="s")
    out_type = jax.ShapeDtypeStruct((NUM_FEATURES * PER_DIM, batch), jnp.float32)
    scratch = (
        [pltpu.VMEM((bpw,), jnp.int32) for _ in range(NUM_FEATURES)]
        + [pltpu.VMEM((bpw, PER_DIM), jnp.float32) for _ in range(NUM_FEATURES)]
        + [pltpu.VMEM((PER_DIM, bpw), jnp.float32) for _ in range(2)]
        + [pltpu.SemaphoreType.DMA, pltpu.SemaphoreType.DMA]
    )

    @functools.partial(
        pl.kernel, mesh=mesh, out_type=out_type, scratch_types=scratch,
        compiler_params=pltpu.CompilerParams(
            use_tc_tiling_on_sc=False, needs_layout_passes=False),
    )
    def gather_kernel(i0, i1, i2, i3, t0, t1, t2, t3, et,
                      x0, x1, x2, x3, r0, r1, r2, r3, c0, c1,
                      sem, osem):
        wid = lax.axis_index("s") * NC + lax.axis_index("c")
        base = wid * bpw
        ids = (i0, i1, i2, i3)
        tabs = (t0, t1, t2, t3)
        idxs = (x0, x1, x2, x3)
        rows = (r0, r1, r2, r3)
        cols = (c0, c1)
        cps = []
        for f in range(NUM_FEATURES):
            pltpu.sync_copy(ids[f].at[pl.ds(base, bpw)], idxs[f])
            cps.append(pltpu.async_copy(tabs[f].at[idxs[f]], rows[f], sem))
        ocps = []
        for f in range(NUM_FEATURES):
            cps[f].wait()
            if f >= 2:
                ocps[f - 2].wait()
            rbuf = rows[f]
            cbuf = cols[f % 2]

            def col_body(c, *, _r=rbuf, _c=cbuf):
                for g in range(bpw // L):
                    ridx = lax.iota(jnp.int32, L) + g * L
                    cidx = jnp.zeros((L,), jnp.int32) + c
                    _c[c, pl.ds(g * L, L)] = plsc.load_gather(_r, [ridx, cidx])

            pl.loop(0, PER_DIM)(col_body)
            ocps.append(pltpu.async_copy(
                cbuf,
                et.at[pl.ds(f * PER_DIM, PER_DIM), pl.ds(base, bpw)],
                osem,
            ))
        ocps[-2].wait()
        ocps[-1].wait()

    return gather_kernel


def _proj_body(et, w, b, o):
    acc = lax.dot_general(
        et[...], w[...],
        dimension_numbers=(((0,), (1,)), ((), ())),
        preferred_element_type=jnp.float32,
    )
    o[...] = acc + b[...]


def _project(et, w, b2d):
    total_dim, batch = et.shape
    out_dim = w.shape[0]
    blk = min(batch, 2048)
    grid = (batch // blk,)
    return pl.pallas_call(
        _proj_body,
        grid=grid,
        in_specs=[pl.BlockSpec((total_dim, blk), lambda i: (0, i)),
                  pl.BlockSpec((out_dim, total_dim), lambda i: (0, 0)),
                  pl.BlockSpec((1, out_dim), lambda i: (0, 0))],
        out_specs=pl.BlockSpec((blk, out_dim), lambda i: (i, 0)),
        out_shape=jax.ShapeDtypeStruct((batch, out_dim), jnp.float32),
    )(et, w, b2d)


def kernel(ids0, ids1, ids2, ids3, T0, T1, T2, T3, W, b):
    batch = ids0.shape[0]
    vocabs = tuple(t.shape[0] for t in (T0, T1, T2, T3))
    gather = _make_gather(batch, vocabs)
    et = gather(ids0, ids1, ids2, ids3, T0, T1, T2, T3)
    b2d = b.reshape(1, -1)
    return _project(et, W, b2d)
